# baseline (speedup 1.0000x reference)
import jax
import jax.numpy as jnp
from jax.experimental import pallas as pl
from jax.experimental.pallas import tpu as pltpu

_STEP = 0.1
_OUTER = 15
_INNER = 40
_N = 384
_RB = 32
_CB = 128


def _shl(a):
    return jnp.concatenate([a[:, 1:], jnp.zeros_like(a[:, :1])], axis=1)


def _shr(a):
    return jnp.concatenate([jnp.zeros_like(a[:, :1]), a[:, :-1]], axis=1)


def _shu(a):
    return jnp.concatenate([a[1:, :], jnp.zeros_like(a[:1, :])], axis=0)


def _shd(a):
    return jnp.concatenate([jnp.zeros_like(a[:1, :]), a[:-1, :]], axis=0)


def _fista(Yb, axis, mask):
    fwd, bwd = (_shl, _shr) if axis == 1 else (_shu, _shd)
    C = 0.25 * (fwd(Yb) - Yb)
    z0 = jnp.zeros_like(Yb)

    def body(_, carry):
        z, w, t = carry
        s = (w + fwd(w)) + (w + bwd(w))
        z_new = jnp.clip(C + 0.25 * s, -_STEP, _STEP) * mask
        t_new = (1.0 + jnp.sqrt(1.0 + 4.0 * t * t)) / 2.0
        w_new = z_new + ((t - 1.0) / t_new) * (z_new - z)
        return (z_new, w_new, t_new)

    z, _, _ = jax.lax.fori_loop(0, _INNER, body, (z0, z0, jnp.float32(1.0)))
    return Yb - (bwd(z) - z)


def _tv2d_kernel(x_ref, o_ref, p_ref, q_ref, y_ref):
    n = _N
    lane = jax.lax.broadcasted_iota(jnp.int32, (_RB, n), 1)
    mask_lane = (lane < n - 1).astype(jnp.float32)
    sub = jax.lax.broadcasted_iota(jnp.int32, (n, _CB), 0)
    mask_sub = (sub < n - 1).astype(jnp.float32)

    o_ref[...] = x_ref[...]
    p_ref[...] = jnp.zeros((n, n), jnp.float32)
    q_ref[...] = jnp.zeros((n, n), jnp.float32)

    def outer(_, carry):
        for j in range(n // _CB):
            sl = slice(j * _CB, (j + 1) * _CB)
            y_ref[:, sl] = _fista(o_ref[:, sl] + p_ref[:, sl], 0, mask_sub)
        p_ref[...] = p_ref[...] + o_ref[...] - y_ref[...]
        for i in range(n // _RB):
            sl = slice(i * _RB, (i + 1) * _RB)
            o_ref[sl, :] = _fista(y_ref[sl, :] + q_ref[sl, :], 1, mask_lane)
        q_ref[...] = q_ref[...] + y_ref[...] - o_ref[...]
        return carry

    jax.lax.fori_loop(0, _OUTER, outer, 0)


@jax.jit
def kernel(x):
    return pl.pallas_call(
        _tv2d_kernel,
        out_shape=jax.ShapeDtypeStruct(x.shape, x.dtype),
        scratch_shapes=[
            pltpu.VMEM((_N, _N), jnp.float32),
            pltpu.VMEM((_N, _N), jnp.float32),
            pltpu.VMEM((_N, _N), jnp.float32),
        ],
    )(x)


# speedup vs baseline: 1.5020x; 1.5020x over previous
import math

import jax
import jax.numpy as jnp
import numpy as np
from jax.experimental import pallas as pl
from jax.experimental.pallas import tpu as pltpu

_STEP = 0.1
_OUTER = 15
_INNER = 40
_N = 384
_RB = 64
_CB = 128


def _momentum_coefs():
    t = 1.0
    coefs = []
    for _ in range(_INNER):
        t_new = (1.0 + math.sqrt(1.0 + 4.0 * t * t)) / 2.0
        coefs.append((t - 1.0) / t_new)
        t = t_new
    return np.asarray(coefs, np.float32)


def _shl(a):
    return jnp.concatenate([a[:, 1:], jnp.zeros_like(a[:, :1])], axis=1)


def _shr(a):
    return jnp.concatenate([jnp.zeros_like(a[:, :1]), a[:, :-1]], axis=1)


def _shu(a):
    return jnp.concatenate([a[1:, :], jnp.zeros_like(a[:1, :])], axis=0)


def _shd(a):
    return jnp.concatenate([jnp.zeros_like(a[:1, :]), a[:-1, :]], axis=0)


def _fista(Yb, axis, mask, coef_ref):
    fwd, bwd = (_shl, _shr) if axis == 1 else (_shu, _shd)
    C = 0.25 * (fwd(Yb) - Yb)
    z0 = jnp.zeros_like(Yb)

    def body(i, carry):
        z, w = carry
        s = (w + fwd(w)) + (w + bwd(w))
        z_new = jnp.clip(C + 0.25 * s, -_STEP, _STEP) * mask
        w_new = z_new + coef_ref[i] * (z_new - z)
        return (z_new, w_new)

    z, _ = jax.lax.fori_loop(0, _INNER, body, (z0, z0))
    return Yb - (bwd(z) - z)


def _tv2d_kernel(coef_ref, x_ref, o_ref, p_ref, q_ref, y_ref):
    n = _N
    lane = jax.lax.broadcasted_iota(jnp.int32, (_RB, n), 1)
    mask_lane = (lane < n - 1).astype(jnp.float32)
    sub = jax.lax.broadcasted_iota(jnp.int32, (n, _CB), 0)
    mask_sub = (sub < n - 1).astype(jnp.float32)

    o_ref[...] = x_ref[...]
    p_ref[...] = jnp.zeros((n, n), jnp.float32)
    q_ref[...] = jnp.zeros((n, n), jnp.float32)

    def outer(_, carry):
        for j in range(n // _CB):
            sl = slice(j * _CB, (j + 1) * _CB)
            y_ref[:, sl] = _fista(
                o_ref[:, sl] + p_ref[:, sl], 0, mask_sub, coef_ref)
        p_ref[...] = p_ref[...] + o_ref[...] - y_ref[...]
        for i in range(n // _RB):
            sl = slice(i * _RB, (i + 1) * _RB)
            o_ref[sl, :] = _fista(
                y_ref[sl, :] + q_ref[sl, :], 1, mask_lane, coef_ref)
        q_ref[...] = q_ref[...] + y_ref[...] - o_ref[...]
        return carry

    jax.lax.fori_loop(0, _OUTER, outer, 0)


@jax.jit
def kernel(x):
    coefs = jnp.asarray(_momentum_coefs())
    return pl.pallas_call(
        _tv2d_kernel,
        out_shape=jax.ShapeDtypeStruct(x.shape, x.dtype),
        in_specs=[
            pl.BlockSpec(memory_space=pltpu.SMEM),
            pl.BlockSpec(memory_space=pltpu.VMEM),
        ],
        scratch_shapes=[
            pltpu.VMEM((_N, _N), jnp.float32),
            pltpu.VMEM((_N, _N), jnp.float32),
            pltpu.VMEM((_N, _N), jnp.float32),
        ],
    )(coefs, x)


# speedup vs baseline: 1.7243x; 1.1480x over previous
import math

import jax
import jax.numpy as jnp
import numpy as np
from jax.experimental import pallas as pl
from jax.experimental.pallas import tpu as pltpu

_STEP = 0.1
_OUTER = 15
_INNER = 40
_N = 384
_RB = 64
_CB = 128


def _momentum_coefs():
    t = 1.0
    coefs = []
    for _ in range(_INNER):
        t_new = (1.0 + math.sqrt(1.0 + 4.0 * t * t)) / 2.0
        coefs.append((t - 1.0) / t_new)
        t = t_new
    return np.asarray(coefs, np.float32)


def _shl(a):
    return jnp.concatenate([a[:, 1:], jnp.zeros_like(a[:, :1])], axis=1)


def _shr(a):
    return jnp.concatenate([jnp.zeros_like(a[:, :1]), a[:, :-1]], axis=1)


def _shu(a):
    return jnp.concatenate([a[1:, :], jnp.zeros_like(a[:1, :])], axis=0)


def _shd(a):
    return jnp.concatenate([jnp.zeros_like(a[:1, :]), a[:-1, :]], axis=0)


def _fista(Yb, axis, mask, coef_ref):
    fwd, bwd = (_shl, _shr) if axis == 1 else (_shu, _shd)
    C = 0.25 * (fwd(Yb) - Yb)
    z0 = jnp.zeros_like(Yb)

    def body(i, carry):
        z, w = carry
        s = (w + fwd(w)) + (w + bwd(w))
        z_new = jnp.clip(C + 0.25 * s, -_STEP, _STEP) * mask
        w_new = z_new + coef_ref[i] * (z_new - z)
        return (z_new, w_new)

    z, _ = jax.lax.fori_loop(0, _INNER, body, (z0, z0), unroll=4)
    return Yb - (bwd(z) - z)


def _tv2d_kernel(coef_ref, x_ref, o_ref, p_ref, q_ref, y_ref):
    n = _N
    lane = jax.lax.broadcasted_iota(jnp.int32, (_RB, n), 1)
    mask_lane = (lane < n - 1).astype(jnp.float32)
    sub = jax.lax.broadcasted_iota(jnp.int32, (n, _CB), 0)
    mask_sub = (sub < n - 1).astype(jnp.float32)

    o_ref[...] = x_ref[...]
    p_ref[...] = jnp.zeros((n, n), jnp.float32)
    q_ref[...] = jnp.zeros((n, n), jnp.float32)

    def outer(_, carry):
        for j in range(n // _CB):
            sl = slice(j * _CB, (j + 1) * _CB)
            y_ref[:, sl] = _fista(
                o_ref[:, sl] + p_ref[:, sl], 0, mask_sub, coef_ref)
        p_ref[...] = p_ref[...] + o_ref[...] - y_ref[...]
        for i in range(n // _RB):
            sl = slice(i * _RB, (i + 1) * _RB)
            o_ref[sl, :] = _fista(
                y_ref[sl, :] + q_ref[sl, :], 1, mask_lane, coef_ref)
        q_ref[...] = q_ref[...] + y_ref[...] - o_ref[...]
        return carry

    jax.lax.fori_loop(0, _OUTER, outer, 0)


@jax.jit
def kernel(x):
    coefs = jnp.asarray(_momentum_coefs())
    return pl.pallas_call(
        _tv2d_kernel,
        out_shape=jax.ShapeDtypeStruct(x.shape, x.dtype),
        in_specs=[
            pl.BlockSpec(memory_space=pltpu.SMEM),
            pl.BlockSpec(memory_space=pltpu.VMEM),
        ],
        scratch_shapes=[
            pltpu.VMEM((_N, _N), jnp.float32),
            pltpu.VMEM((_N, _N), jnp.float32),
            pltpu.VMEM((_N, _N), jnp.float32),
        ],
    )(coefs, x)


# speedup vs baseline: 1.8895x; 1.0958x over previous
import math

import jax
import jax.numpy as jnp
import numpy as np
from jax.experimental import pallas as pl
from jax.experimental.pallas import tpu as pltpu

_STEP = 0.1
_OUTER = 15
_INNER = 40
_N = 384
_RB = 64
_CB = 128


def _momentum_coefs():
    t = 1.0
    coefs = []
    for _ in range(_INNER):
        t_new = (1.0 + math.sqrt(1.0 + 4.0 * t * t)) / 2.0
        coefs.append((t - 1.0) / t_new)
        t = t_new
    return np.asarray(coefs, np.float32)


def _shl(a):
    return jnp.concatenate([a[:, 1:], jnp.zeros_like(a[:, :1])], axis=1)


def _shr(a):
    return jnp.concatenate([jnp.zeros_like(a[:, :1]), a[:, :-1]], axis=1)


def _shu(a):
    return jnp.concatenate([a[1:, :], jnp.zeros_like(a[:1, :])], axis=0)


def _shd(a):
    return jnp.concatenate([jnp.zeros_like(a[:1, :]), a[:-1, :]], axis=0)


def _fista(Yb, axis, mask, coef_ref):
    fwd, bwd = (_shl, _shr) if axis == 1 else (_shu, _shd)
    C = 0.25 * (fwd(Yb) - Yb)
    z0 = jnp.zeros_like(Yb)

    def body(i, carry):
        z, w = carry
        s = (w + fwd(w)) + (w + bwd(w))
        z_new = jnp.clip(C + 0.25 * s, -_STEP, _STEP) * mask
        w_new = z_new + coef_ref[i] * (z_new - z)
        return (z_new, w_new)

    z, _ = jax.lax.fori_loop(0, _INNER, body, (z0, z0), unroll=8)
    return Yb - (bwd(z) - z)


def _tv2d_kernel(coef_ref, x_ref, o_ref, p_ref, q_ref, y_ref):
    n = _N
    lane = jax.lax.broadcasted_iota(jnp.int32, (_RB, n), 1)
    mask_lane = (lane < n - 1).astype(jnp.float32)
    sub = jax.lax.broadcasted_iota(jnp.int32, (n, _CB), 0)
    mask_sub = (sub < n - 1).astype(jnp.float32)

    o_ref[...] = x_ref[...]
    p_ref[...] = jnp.zeros((n, n), jnp.float32)
    q_ref[...] = jnp.zeros((n, n), jnp.float32)

    def outer(_, carry):
        for j in range(n // _CB):
            sl = slice(j * _CB, (j + 1) * _CB)
            y_ref[:, sl] = _fista(
                o_ref[:, sl] + p_ref[:, sl], 0, mask_sub, coef_ref)
        p_ref[...] = p_ref[...] + o_ref[...] - y_ref[...]
        for i in range(n // _RB):
            sl = slice(i * _RB, (i + 1) * _RB)
            o_ref[sl, :] = _fista(
                y_ref[sl, :] + q_ref[sl, :], 1, mask_lane, coef_ref)
        q_ref[...] = q_ref[...] + y_ref[...] - o_ref[...]
        return carry

    jax.lax.fori_loop(0, _OUTER, outer, 0)


@jax.jit
def kernel(x):
    coefs = jnp.asarray(_momentum_coefs())
    return pl.pallas_call(
        _tv2d_kernel,
        out_shape=jax.ShapeDtypeStruct(x.shape, x.dtype),
        in_specs=[
            pl.BlockSpec(memory_space=pltpu.SMEM),
            pl.BlockSpec(memory_space=pltpu.VMEM),
        ],
        scratch_shapes=[
            pltpu.VMEM((_N, _N), jnp.float32),
            pltpu.VMEM((_N, _N), jnp.float32),
            pltpu.VMEM((_N, _N), jnp.float32),
        ],
    )(coefs, x)


# speedup vs baseline: 1.9844x; 1.0502x over previous
import math

import jax
import jax.numpy as jnp
import numpy as np
from jax.experimental import pallas as pl
from jax.experimental.pallas import tpu as pltpu

_STEP = 0.1
_OUTER = 15
_INNER = 40
_N = 384
_RB = 64
_CB = 128


def _momentum_coefs():
    t = 1.0
    coefs = []
    for _ in range(_INNER):
        t_new = (1.0 + math.sqrt(1.0 + 4.0 * t * t)) / 2.0
        coefs.append((t - 1.0) / t_new)
        t = t_new
    return np.asarray(coefs, np.float32)


def _shl(a):
    return jnp.concatenate([a[:, 1:], jnp.zeros_like(a[:, :1])], axis=1)


def _shr(a):
    return jnp.concatenate([jnp.zeros_like(a[:, :1]), a[:, :-1]], axis=1)


def _shu(a):
    return jnp.concatenate([a[1:, :], jnp.zeros_like(a[:1, :])], axis=0)


def _shd(a):
    return jnp.concatenate([jnp.zeros_like(a[:1, :]), a[:-1, :]], axis=0)


def _fista(Yb, axis, mask, coef_ref):
    fwd, bwd = (_shl, _shr) if axis == 1 else (_shu, _shd)
    C = 0.25 * (fwd(Yb) - Yb)
    z0 = jnp.zeros_like(Yb)

    def body(i, carry):
        z, w = carry
        s = (w + fwd(w)) + (w + bwd(w))
        z_new = jnp.clip(C + 0.25 * s, -_STEP, _STEP) * mask
        w_new = z_new + coef_ref[i] * (z_new - z)
        return (z_new, w_new)

    z, _ = jax.lax.fori_loop(0, _INNER, body, (z0, z0), unroll=20)
    return Yb - (bwd(z) - z)


def _tv2d_kernel(coef_ref, x_ref, o_ref, p_ref, q_ref, y_ref):
    n = _N
    lane = jax.lax.broadcasted_iota(jnp.int32, (_RB, n), 1)
    mask_lane = (lane < n - 1).astype(jnp.float32)
    sub = jax.lax.broadcasted_iota(jnp.int32, (n, _CB), 0)
    mask_sub = (sub < n - 1).astype(jnp.float32)

    o_ref[...] = x_ref[...]
    p_ref[...] = jnp.zeros((n, n), jnp.float32)
    q_ref[...] = jnp.zeros((n, n), jnp.float32)

    def outer(_, carry):
        for j in range(n // _CB):
            sl = slice(j * _CB, (j + 1) * _CB)
            y_ref[:, sl] = _fista(
                o_ref[:, sl] + p_ref[:, sl], 0, mask_sub, coef_ref)
        p_ref[...] = p_ref[...] + o_ref[...] - y_ref[...]
        for i in range(n // _RB):
            sl = slice(i * _RB, (i + 1) * _RB)
            o_ref[sl, :] = _fista(
                y_ref[sl, :] + q_ref[sl, :], 1, mask_lane, coef_ref)
        q_ref[...] = q_ref[...] + y_ref[...] - o_ref[...]
        return carry

    jax.lax.fori_loop(0, _OUTER, outer, 0)


@jax.jit
def kernel(x):
    coefs = jnp.asarray(_momentum_coefs())
    return pl.pallas_call(
        _tv2d_kernel,
        out_shape=jax.ShapeDtypeStruct(x.shape, x.dtype),
        in_specs=[
            pl.BlockSpec(memory_space=pltpu.SMEM),
            pl.BlockSpec(memory_space=pltpu.VMEM),
        ],
        scratch_shapes=[
            pltpu.VMEM((_N, _N), jnp.float32),
            pltpu.VMEM((_N, _N), jnp.float32),
            pltpu.VMEM((_N, _N), jnp.float32),
        ],
    )(coefs, x)


# speedup vs baseline: 2.2250x; 1.1213x over previous
import math

import jax
import jax.numpy as jnp
import numpy as np
from jax.experimental import pallas as pl
from jax.experimental.pallas import tpu as pltpu

_STEP = 0.1
_OUTER = 15
_INNER = 40
_N = 384
_RB = 64
_CB = 128


def _momentum_coefs():
    t = 1.0
    coefs = []
    for _ in range(_INNER):
        t_new = (1.0 + math.sqrt(1.0 + 4.0 * t * t)) / 2.0
        coefs.append((t - 1.0) / t_new)
        t = t_new
    return np.asarray(coefs, np.float32)


def _shl(a):
    return jnp.concatenate([a[:, 1:], jnp.zeros_like(a[:, :1])], axis=1)


def _shr(a):
    return jnp.concatenate([jnp.zeros_like(a[:, :1]), a[:, :-1]], axis=1)


def _shu(a):
    return jnp.concatenate([a[1:, :], jnp.zeros_like(a[:1, :])], axis=0)


def _shd(a):
    return jnp.concatenate([jnp.zeros_like(a[:1, :]), a[:-1, :]], axis=0)


def _fista(Yb, axis, mask, coef_ref):
    fwd, bwd = (_shl, _shr) if axis == 1 else (_shu, _shd)
    C = 0.25 * (fwd(Yb) - Yb)
    z0 = jnp.zeros_like(Yb)

    def body(i, carry):
        z, w = carry
        s = (w + fwd(w)) + (w + bwd(w))
        z_new = jnp.clip(C + 0.25 * s, -_STEP, _STEP) * mask
        w_new = z_new + coef_ref[i] * (z_new - z)
        return (z_new, w_new)

    z, _ = jax.lax.fori_loop(0, _INNER, body, (z0, z0), unroll=40)
    return Yb - (bwd(z) - z)


def _tv2d_kernel(coef_ref, x_ref, o_ref, p_ref, q_ref, y_ref):
    n = _N
    lane = jax.lax.broadcasted_iota(jnp.int32, (_RB, n), 1)
    mask_lane = (lane < n - 1).astype(jnp.float32)
    sub = jax.lax.broadcasted_iota(jnp.int32, (n, _CB), 0)
    mask_sub = (sub < n - 1).astype(jnp.float32)

    o_ref[...] = x_ref[...]
    p_ref[...] = jnp.zeros((n, n), jnp.float32)
    q_ref[...] = jnp.zeros((n, n), jnp.float32)

    def outer(_, carry):
        for j in range(n // _CB):
            sl = slice(j * _CB, (j + 1) * _CB)
            y_ref[:, sl] = _fista(
                o_ref[:, sl] + p_ref[:, sl], 0, mask_sub, coef_ref)
        p_ref[...] = p_ref[...] + o_ref[...] - y_ref[...]
        for i in range(n // _RB):
            sl = slice(i * _RB, (i + 1) * _RB)
            o_ref[sl, :] = _fista(
                y_ref[sl, :] + q_ref[sl, :], 1, mask_lane, coef_ref)
        q_ref[...] = q_ref[...] + y_ref[...] - o_ref[...]
        return carry

    jax.lax.fori_loop(0, _OUTER, outer, 0)


@jax.jit
def kernel(x):
    coefs = jnp.asarray(_momentum_coefs())
    return pl.pallas_call(
        _tv2d_kernel,
        out_shape=jax.ShapeDtypeStruct(x.shape, x.dtype),
        in_specs=[
            pl.BlockSpec(memory_space=pltpu.SMEM),
            pl.BlockSpec(memory_space=pltpu.VMEM),
        ],
        scratch_shapes=[
            pltpu.VMEM((_N, _N), jnp.float32),
            pltpu.VMEM((_N, _N), jnp.float32),
            pltpu.VMEM((_N, _N), jnp.float32),
        ],
    )(coefs, x)


# speedup vs baseline: 2.2291x; 1.0018x over previous
import math

import jax
import jax.numpy as jnp
import numpy as np
from jax.experimental import pallas as pl
from jax.experimental.pallas import tpu as pltpu

_STEP = 0.1
_OUTER = 15
_INNER = 40
_N = 384
_RB = 64
_CB = 128


def _momentum_coefs():
    t = 1.0
    coefs = []
    for _ in range(_INNER):
        t_new = (1.0 + math.sqrt(1.0 + 4.0 * t * t)) / 2.0
        coefs.append((t - 1.0) / t_new)
        t = t_new
    return np.asarray(coefs, np.float32)


def _shl(a):
    return jnp.concatenate([a[:, 1:], jnp.zeros_like(a[:, :1])], axis=1)


def _shr(a):
    return jnp.concatenate([jnp.zeros_like(a[:, :1]), a[:, :-1]], axis=1)


def _shu(a):
    return jnp.concatenate([a[1:, :], jnp.zeros_like(a[:1, :])], axis=0)


def _shd(a):
    return jnp.concatenate([jnp.zeros_like(a[:1, :]), a[:-1, :]], axis=0)


_COEFS = _momentum_coefs()


def _fista(Yb, axis, mask):
    fwd, bwd = (_shl, _shr) if axis == 1 else (_shu, _shd)
    C = 0.25 * (fwd(Yb) - Yb)
    z = jnp.zeros_like(Yb)
    w = z

    for i in range(_INNER):
        s = (w + fwd(w)) + (w + bwd(w))
        z_new = jnp.clip(C + 0.25 * s, -_STEP, _STEP) * mask
        w = z_new + float(_COEFS[i]) * (z_new - z)
        z = z_new

    return Yb - (bwd(z) - z)


def _tv2d_kernel(x_ref, o_ref, p_ref, q_ref, y_ref):
    n = _N
    lane = jax.lax.broadcasted_iota(jnp.int32, (_RB, n), 1)
    mask_lane = (lane < n - 1).astype(jnp.float32)
    sub = jax.lax.broadcasted_iota(jnp.int32, (n, _CB), 0)
    mask_sub = (sub < n - 1).astype(jnp.float32)

    o_ref[...] = x_ref[...]
    p_ref[...] = jnp.zeros((n, n), jnp.float32)
    q_ref[...] = jnp.zeros((n, n), jnp.float32)

    def outer(_, carry):
        for j in range(n // _CB):
            sl = slice(j * _CB, (j + 1) * _CB)
            y_ref[:, sl] = _fista(
                o_ref[:, sl] + p_ref[:, sl], 0, mask_sub)
        p_ref[...] = p_ref[...] + o_ref[...] - y_ref[...]
        for i in range(n // _RB):
            sl = slice(i * _RB, (i + 1) * _RB)
            o_ref[sl, :] = _fista(
                y_ref[sl, :] + q_ref[sl, :], 1, mask_lane)
        q_ref[...] = q_ref[...] + y_ref[...] - o_ref[...]
        return carry

    jax.lax.fori_loop(0, _OUTER, outer, 0)


@jax.jit
def kernel(x):
    return pl.pallas_call(
        _tv2d_kernel,
        out_shape=jax.ShapeDtypeStruct(x.shape, x.dtype),
        in_specs=[
            pl.BlockSpec(memory_space=pltpu.VMEM),
        ],
        scratch_shapes=[
            pltpu.VMEM((_N, _N), jnp.float32),
            pltpu.VMEM((_N, _N), jnp.float32),
            pltpu.VMEM((_N, _N), jnp.float32),
        ],
    )(x)


# speedup vs baseline: 3.0883x; 1.3854x over previous
import math

import jax
import jax.numpy as jnp
import numpy as np
from jax.experimental import pallas as pl
from jax.experimental.pallas import tpu as pltpu

_STEP = 0.1
_OUTER = 15
_INNER = 40
_N = 384
_RB = 64
_CB = 128


def _momentum_coefs():
    t = 1.0
    coefs = []
    for _ in range(_INNER):
        t_new = (1.0 + math.sqrt(1.0 + 4.0 * t * t)) / 2.0
        coefs.append((t - 1.0) / t_new)
        t = t_new
    return np.asarray(coefs, np.float32)


def _shl(a):
    return jnp.concatenate([a[:, 1:], jnp.zeros_like(a[:, :1])], axis=1)


def _shr(a):
    return jnp.concatenate([jnp.zeros_like(a[:, :1]), a[:, :-1]], axis=1)


def _shu(a):
    return jnp.concatenate([a[1:, :], jnp.zeros_like(a[:1, :])], axis=0)


def _shd(a):
    return jnp.concatenate([jnp.zeros_like(a[:1, :]), a[:-1, :]], axis=0)


_COEFS = _momentum_coefs()


def _fista(Ybs, axis, mask):
    fwd, bwd = (_shl, _shr) if axis == 1 else (_shu, _shd)
    Cs = [0.25 * (fwd(Yb) - Yb) for Yb in Ybs]
    zs = [jnp.zeros_like(Yb) for Yb in Ybs]
    ws = list(zs)

    for i in range(_INNER):
        c = float(_COEFS[i])
        for k in range(len(Ybs)):
            w, z = ws[k], zs[k]
            s = (w + fwd(w)) + (w + bwd(w))
            z_new = jnp.clip(Cs[k] + 0.25 * s, -_STEP, _STEP) * mask
            ws[k] = z_new + c * (z_new - z)
            zs[k] = z_new

    return [Yb - (bwd(z) - z) for Yb, z in zip(Ybs, zs)]


def _tv2d_kernel(x_ref, o_ref, p_ref, q_ref, y_ref):
    n = _N
    lane = jax.lax.broadcasted_iota(jnp.int32, (_RB, n), 1)
    mask_lane = (lane < n - 1).astype(jnp.float32)
    sub = jax.lax.broadcasted_iota(jnp.int32, (n, _CB), 0)
    mask_sub = (sub < n - 1).astype(jnp.float32)

    o_ref[...] = x_ref[...]
    p_ref[...] = jnp.zeros((n, n), jnp.float32)
    q_ref[...] = jnp.zeros((n, n), jnp.float32)

    col_slices = [slice(j * _CB, (j + 1) * _CB) for j in range(n // _CB)]
    row_slices = [slice(i * _RB, (i + 1) * _RB) for i in range(n // _RB)]

    def grouped(slices, group):
        return [slices[i:i + group] for i in range(0, len(slices), group)]

    def outer(_, carry):
        for grp in grouped(col_slices, 2):
            outs = _fista(
                [o_ref[:, sl] + p_ref[:, sl] for sl in grp], 0, mask_sub)
            for sl, ob in zip(grp, outs):
                y_ref[:, sl] = ob
        p_ref[...] = p_ref[...] + o_ref[...] - y_ref[...]
        for grp in grouped(row_slices, 2):
            outs = _fista(
                [y_ref[sl, :] + q_ref[sl, :] for sl in grp], 1, mask_lane)
            for sl, ob in zip(grp, outs):
                o_ref[sl, :] = ob
        q_ref[...] = q_ref[...] + y_ref[...] - o_ref[...]
        return carry

    jax.lax.fori_loop(0, _OUTER, outer, 0)


@jax.jit
def kernel(x):
    return pl.pallas_call(
        _tv2d_kernel,
        out_shape=jax.ShapeDtypeStruct(x.shape, x.dtype),
        in_specs=[
            pl.BlockSpec(memory_space=pltpu.VMEM),
        ],
        scratch_shapes=[
            pltpu.VMEM((_N, _N), jnp.float32),
            pltpu.VMEM((_N, _N), jnp.float32),
            pltpu.VMEM((_N, _N), jnp.float32),
        ],
    )(x)


# speedup vs baseline: 3.1912x; 1.0333x over previous
import math

import jax
import jax.numpy as jnp
import numpy as np
from jax.experimental import pallas as pl
from jax.experimental.pallas import tpu as pltpu

_STEP = 0.1
_OUTER = 15
_INNER = 40
_N = 384
_RB = 64
_CB = 128


def _momentum_coefs():
    t = 1.0
    coefs = []
    for _ in range(_INNER):
        t_new = (1.0 + math.sqrt(1.0 + 4.0 * t * t)) / 2.0
        coefs.append((t - 1.0) / t_new)
        t = t_new
    return np.asarray(coefs, np.float32)


def _shl(a):
    return jnp.concatenate([a[:, 1:], jnp.zeros_like(a[:, :1])], axis=1)


def _shr(a):
    return jnp.concatenate([jnp.zeros_like(a[:, :1]), a[:, :-1]], axis=1)


def _shu(a):
    return jnp.concatenate([a[1:, :], jnp.zeros_like(a[:1, :])], axis=0)


def _shd(a):
    return jnp.concatenate([jnp.zeros_like(a[:1, :]), a[:-1, :]], axis=0)


_COEFS = _momentum_coefs()


def _fista(Ybs, axis, mask):
    fwd, bwd = (_shl, _shr) if axis == 1 else (_shu, _shd)
    Cs = [0.25 * (fwd(Yb) - Yb) for Yb in Ybs]
    zs = [jnp.zeros_like(Yb) for Yb in Ybs]
    ws = list(zs)

    for i in range(_INNER):
        c = float(_COEFS[i])
        for k in range(len(Ybs)):
            w, z = ws[k], zs[k]
            s = (w + fwd(w)) + (w + bwd(w))
            z_new = jnp.clip(Cs[k] + 0.25 * s, -_STEP, _STEP) * mask
            ws[k] = z_new + c * (z_new - z)
            zs[k] = z_new

    return [Yb - (bwd(z) - z) for Yb, z in zip(Ybs, zs)]


def _tv2d_kernel(x_ref, o_ref, p_ref, q_ref, y_ref):
    n = _N
    lane = jax.lax.broadcasted_iota(jnp.int32, (_RB, n), 1)
    mask_lane = (lane < n - 1).astype(jnp.float32)
    sub = jax.lax.broadcasted_iota(jnp.int32, (n, _CB), 0)
    mask_sub = (sub < n - 1).astype(jnp.float32)

    o_ref[...] = x_ref[...]
    p_ref[...] = jnp.zeros((n, n), jnp.float32)
    q_ref[...] = jnp.zeros((n, n), jnp.float32)

    col_slices = [slice(j * _CB, (j + 1) * _CB) for j in range(n // _CB)]
    row_slices = [slice(i * _RB, (i + 1) * _RB) for i in range(n // _RB)]

    def grouped(slices, group):
        return [slices[i:i + group] for i in range(0, len(slices), group)]

    def outer(_, carry):
        for grp in grouped(col_slices, 3):
            outs = _fista(
                [o_ref[:, sl] + p_ref[:, sl] for sl in grp], 0, mask_sub)
            for sl, ob in zip(grp, outs):
                y_ref[:, sl] = ob
        p_ref[...] = p_ref[...] + o_ref[...] - y_ref[...]
        for grp in grouped(row_slices, 3):
            outs = _fista(
                [y_ref[sl, :] + q_ref[sl, :] for sl in grp], 1, mask_lane)
            for sl, ob in zip(grp, outs):
                o_ref[sl, :] = ob
        q_ref[...] = q_ref[...] + y_ref[...] - o_ref[...]
        return carry

    jax.lax.fori_loop(0, _OUTER, outer, 0)


@jax.jit
def kernel(x):
    return pl.pallas_call(
        _tv2d_kernel,
        out_shape=jax.ShapeDtypeStruct(x.shape, x.dtype),
        in_specs=[
            pl.BlockSpec(memory_space=pltpu.VMEM),
        ],
        scratch_shapes=[
            pltpu.VMEM((_N, _N), jnp.float32),
            pltpu.VMEM((_N, _N), jnp.float32),
            pltpu.VMEM((_N, _N), jnp.float32),
        ],
    )(x)


# speedup vs baseline: 3.1991x; 1.0025x over previous
import math

import jax
import jax.numpy as jnp
import numpy as np
from jax.experimental import pallas as pl
from jax.experimental.pallas import tpu as pltpu

_STEP = 0.1
_OUTER = 15
_INNER = 40
_N = 384
_RB = 64
_CB = 128


def _momentum_coefs():
    t = 1.0
    coefs = []
    for _ in range(_INNER):
        t_new = (1.0 + math.sqrt(1.0 + 4.0 * t * t)) / 2.0
        coefs.append((t - 1.0) / t_new)
        t = t_new
    return np.asarray(coefs, np.float32)


def _shl(a):
    return jnp.concatenate([a[:, 1:], jnp.zeros_like(a[:, :1])], axis=1)


def _shr(a):
    return jnp.concatenate([jnp.zeros_like(a[:, :1]), a[:, :-1]], axis=1)


def _shu(a):
    return jnp.concatenate([a[1:, :], jnp.zeros_like(a[:1, :])], axis=0)


def _shd(a):
    return jnp.concatenate([jnp.zeros_like(a[:1, :]), a[:-1, :]], axis=0)


_COEFS = _momentum_coefs()


def _fista(Ybs, axis, mask):
    fwd, bwd = (_shl, _shr) if axis == 1 else (_shu, _shd)
    Cs = [0.25 * (fwd(Yb) - Yb) for Yb in Ybs]
    zs = [jnp.zeros_like(Yb) for Yb in Ybs]
    ws = list(zs)

    for i in range(_INNER):
        c = float(_COEFS[i])
        for k in range(len(Ybs)):
            w, z = ws[k], zs[k]
            s = (w + fwd(w)) + (w + bwd(w))
            z_new = jnp.clip(Cs[k] + 0.25 * s, -_STEP, _STEP) * mask
            ws[k] = z_new + c * (z_new - z)
            zs[k] = z_new

    return [Yb - (bwd(z) - z) for Yb, z in zip(Ybs, zs)]


def _tv2d_kernel(x_ref, o_ref, p_ref, q_ref, y_ref):
    n = _N
    lane = jax.lax.broadcasted_iota(jnp.int32, (_RB, n), 1)
    mask_lane = (lane < n - 1).astype(jnp.float32)
    sub = jax.lax.broadcasted_iota(jnp.int32, (n, _CB), 0)
    mask_sub = (sub < n - 1).astype(jnp.float32)

    o_ref[...] = x_ref[...]
    p_ref[...] = jnp.zeros((n, n), jnp.float32)
    q_ref[...] = jnp.zeros((n, n), jnp.float32)

    col_slices = [slice(j * _CB, (j + 1) * _CB) for j in range(n // _CB)]
    row_slices = [slice(i * _RB, (i + 1) * _RB) for i in range(n // _RB)]

    def grouped(slices, group):
        return [slices[i:i + group] for i in range(0, len(slices), group)]

    def outer(_, carry):
        for grp in grouped(col_slices, 3):
            outs = _fista(
                [o_ref[:, sl] + p_ref[:, sl] for sl in grp], 0, mask_sub)
            for sl, ob in zip(grp, outs):
                y_ref[:, sl] = ob
        p_ref[...] = p_ref[...] + o_ref[...] - y_ref[...]
        for grp in grouped(row_slices, 6):
            outs = _fista(
                [y_ref[sl, :] + q_ref[sl, :] for sl in grp], 1, mask_lane)
            for sl, ob in zip(grp, outs):
                o_ref[sl, :] = ob
        q_ref[...] = q_ref[...] + y_ref[...] - o_ref[...]
        return carry

    jax.lax.fori_loop(0, _OUTER, outer, 0)


@jax.jit
def kernel(x):
    return pl.pallas_call(
        _tv2d_kernel,
        out_shape=jax.ShapeDtypeStruct(x.shape, x.dtype),
        in_specs=[
            pl.BlockSpec(memory_space=pltpu.VMEM),
        ],
        scratch_shapes=[
            pltpu.VMEM((_N, _N), jnp.float32),
            pltpu.VMEM((_N, _N), jnp.float32),
            pltpu.VMEM((_N, _N), jnp.float32),
        ],
    )(x)
